# baseline (speedup 1.0000x reference)
import functools

import numpy as np
import jax
import jax.numpy as jnp
from jax import lax
from jax.experimental import pallas as pl
from jax.experimental.pallas import tpu as pltpu
from jax.experimental.pallas import tpu_sc as plsc

N_STAGES = 8


_ROT_A = (13, 15, 26, 6)
_ROT_B = (17, 29, 16, 24)


def _np_threefry(k0, k1, x0, x1):
    M = np.uint64(0xFFFFFFFF)

    def rotl(v, r):
        return ((v << np.uint64(r)) | (v >> np.uint64(32 - r))) & M

    k0 = np.uint64(k0)
    k1 = np.uint64(k1)
    ks2 = (k0 ^ k1 ^ np.uint64(0x1BD11BDA)) & M
    ks = (k0, k1, ks2)
    x0 = (np.uint64(x0) + k0) & M
    x1 = (np.uint64(x1) + k1) & M
    rots = (_ROT_A, _ROT_B)
    for g in range(5):
        for r in rots[g % 2]:
            x0 = (x0 + x1) & M
            x1 = rotl(x1, r)
            x1 ^= x0
        x0 = (x0 + ks[(g + 1) % 3]) & M
        x1 = (x1 + ks[(g + 2) % 3] + np.uint64(g + 1)) & M
    return int(x0), int(x1)


def _fold_key(i):
    return _np_threefry(0, 1234, 0, i)


_KEY_U0 = _fold_key(0)
_KEYS_SYMP = {i: _fold_key(i) for i in range(2, N_STAGES - 1)}
_KEYS_REC = {i: _fold_key(100 + i) for i in range(2, N_STAGES - 1)}

_F32_ONE_BITS = np.uint32(0x3F800000)
_U_LO = np.float32(np.nextafter(np.float32(-1.0), np.float32(0.0)))
_SQRT2 = np.float32(np.sqrt(2.0))


def _rotl(x, r):
    return (x << np.uint32(r)) | (x >> np.uint32(32 - r))


def _cipher_xor(k0, k1, ctr):
    if isinstance(k0, int):
        k0 = jnp.uint32(k0)
    if isinstance(k1, int):
        k1 = jnp.uint32(k1)
    ks2 = k0 ^ k1 ^ jnp.uint32(0x1BD11BDA)
    ks = (k0, k1, ks2)
    x0 = jnp.broadcast_to(k0, ctr.shape).astype(jnp.uint32)
    x1 = ctr + k1
    rots = (_ROT_A, _ROT_B)
    for g in range(5):
        for r in rots[g % 2]:
            x0 = x0 + x1
            x1 = _rotl(x1, r)
            x1 = x1 ^ x0
        x0 = x0 + ks[(g + 1) % 3]
        x1 = x1 + ks[(g + 2) % 3] + jnp.uint32(g + 1)
    return x0 ^ x1


def _bits_to_f01(bits):
    fb = (bits >> jnp.uint32(9)) | _F32_ONE_BITS
    return lax.bitcast_convert_type(fb, jnp.float32) - jnp.float32(1.0)


def _erfinv_f32(x):
    w = -jnp.log((jnp.float32(1.0) - x) * (jnp.float32(1.0) + x))
    w1 = w - jnp.float32(2.5)
    p = jnp.float32(2.81022636e-08)
    for c in (3.43273939e-07, -3.5233877e-06, -4.39150654e-06, 0.00021858087,
              -0.00125372503, -0.00417768164, 0.246640727, 1.50140941):
        p = jnp.float32(c) + p * w1
    w2 = jnp.sqrt(w) - jnp.float32(3.0)
    q = jnp.float32(-0.000200214257)
    for c in (0.000100950558, 0.00134934322, -0.00367342844, 0.00573950773,
              -0.0076224613, 0.00943887047, 1.00167406, 2.83297682):
        q = jnp.float32(c) + q * w2
    return jnp.where(w < jnp.float32(5.0), p, q) * x


def _chain(ist, vals, init_dtype):
    acc = jnp.full(ist.shape, vals[2], init_dtype)
    for i in range(3, N_STAGES - 1):
        acc = jnp.where(ist == i, jnp.asarray(vals[i], init_dtype), acc)
    return acc


def _ctr_for_block(shape, block_elems, blk_off=0):
    blk = pl.program_id(0).astype(jnp.uint32) + jnp.uint32(blk_off)
    row = lax.broadcasted_iota(jnp.uint32, shape, 0)
    col = lax.broadcasted_iota(jnp.uint32, shape, 1)
    return blk * jnp.uint32(block_elems) + row * jnp.uint32(shape[1]) + col


def _tca_body(block_elems, u0_out):
    ctr = _ctr_for_block(u0_out.shape, block_elems)
    u0_out[...] = jnp.maximum(
        jnp.float32(0.0),
        _bits_to_f01(_cipher_xor(_KEY_U0[0], _KEY_U0[1], ctr)))


def _tcb_body(block_elems, blk_off, t_smem, par_ref, cs_ref, ns0_ref,
              tns0_ref, u0_ref, probs_ref, cs1_out, ns_out, tns_out):
    shape = cs_ref.shape
    t = t_smem[0]
    csf = cs_ref[...].astype(jnp.float32)
    ns0 = ns0_ref[...].astype(jnp.float32)
    tns0 = tns0_ref[...]

    mt = (t >= tns0) & (csf < jnp.float32(N_STAGES - 1))
    cs1 = jnp.where(mt, ns0, csf)
    ist = cs1.astype(jnp.int32)

    symp = u0_ref[...] < probs_ref[...]
    upd = mt & (cs1 >= jnp.float32(2.0)) & (cs1 <= jnp.float32(N_STAGES - 2))

    k0s = _chain(ist, {i: np.uint32(k[0]) for i, k in _KEYS_SYMP.items()},
                 jnp.uint32)
    k1s = _chain(ist, {i: np.uint32(k[1]) for i, k in _KEYS_SYMP.items()},
                 jnp.uint32)
    k0r = _chain(ist, {i: np.uint32(k[0]) for i, k in _KEYS_REC.items()},
                 jnp.uint32)
    k1r = _chain(ist, {i: np.uint32(k[1]) for i, k in _KEYS_REC.items()},
                 jnp.uint32)
    k0 = jnp.where(symp, k0s, k0r)
    k1 = jnp.where(symp, k1s, k1r)

    ctr = _ctr_for_block(shape, block_elems, blk_off)
    f = _bits_to_f01(_cipher_xor(k0, k1, ctr))
    u = f * jnp.float32(2.0) + _U_LO
    u = jnp.maximum(_U_LO, u)
    eps = _SQRT2 * _erfinv_f32(u)

    def chain_par(row_idx):
        acc = jnp.full(shape, par_ref[row_idx, 2])
        for i in range(3, N_STAGES - 1):
            acc = jnp.where(ist == i, par_ref[row_idx, i], acc)
        return acc

    mu = jnp.where(symp, chain_par(0), chain_par(2))
    sig = jnp.where(symp, chain_par(1), chain_par(3))
    samp = jnp.exp(mu + sig * eps)

    cs1_out[...] = cs1
    ns_out[...] = jnp.where(upd & symp, ns0 + jnp.float32(1.0),
                            jnp.where(upd, jnp.float32(0.0), ns0))
    tns_out[...] = jnp.where(upd, tns0 + samp, tns0)


def _tc_a(shape2, *, block_rows, interpret=False):
    nrows, ncols = shape2
    grid = nrows // block_rows
    bspec = pl.BlockSpec((block_rows, ncols), lambda i: (i, 0))
    f32 = jax.ShapeDtypeStruct((nrows, ncols), jnp.float32)
    return pl.pallas_call(
        functools.partial(_tca_body, block_rows * ncols),
        grid=(grid,),
        in_specs=[],
        out_specs=[bspec],
        out_shape=[f32],
        compiler_params=pltpu.CompilerParams(
            dimension_semantics=("arbitrary",)),
        interpret=interpret,
    )()[0]


def _tc_b(cs2, ns02, tns02, u0, probs2, t1, params, *, block_rows,
          blk_off=0, n_blocks=None, interpret=False):
    nrows, ncols = cs2.shape
    grid = n_blocks if n_blocks is not None else nrows // block_rows
    bspec = pl.BlockSpec((block_rows, ncols), lambda i: (i + blk_off, 0))
    f32 = jax.ShapeDtypeStruct((nrows, ncols), jnp.float32)
    smem = pl.BlockSpec(memory_space=pltpu.SMEM)
    return pl.pallas_call(
        functools.partial(_tcb_body, block_rows * ncols, blk_off),
        grid=(grid,),
        in_specs=[smem, smem, bspec, bspec, bspec, bspec, bspec],
        out_specs=[bspec, bspec, bspec],
        out_shape=[f32, f32, f32],
        compiler_params=pltpu.CompilerParams(
            dimension_semantics=("arbitrary",)),
        interpret=interpret,
    )(t1, params, cs2, ns02, tns02, u0, probs2)


def _sc_gather(cs_p, ns0_p, tns0_p, ages_p, table_flat, t16, npad,
               chunk_lo, n_chunks, tail_off, tail_len):
    info = plsc.get_sparse_core_info()
    nw = info.num_cores * info.num_subcores
    chunk = 8192
    tbl_n = table_flat.shape[0]
    mesh = plsc.VectorSubcoreMesh(core_axis_name="c", subcore_axis_name="s")

    @functools.partial(
        pl.kernel, mesh=mesh,
        compiler_params=pltpu.CompilerParams(needs_layout_passes=False),
        out_type=jax.ShapeDtypeStruct((npad,), jnp.float32),
        scratch_types=[
            pltpu.VMEM((tbl_n,), jnp.float32),
            pltpu.VMEM((16,), jnp.float32),
            [pltpu.VMEM((chunk,), jnp.int32) for _ in range(2)],
            [pltpu.VMEM((chunk,), jnp.int32) for _ in range(2)],
            [pltpu.VMEM((chunk,), jnp.float32) for _ in range(2)],
            [pltpu.VMEM((chunk,), jnp.int32) for _ in range(2)],
            [pltpu.VMEM((chunk,), jnp.float32) for _ in range(2)],
            [pltpu.SemaphoreType.DMA for _ in range(2)],
        ],
    )
    def sc_k(cs_hbm, ns0_hbm, tns0_hbm, ages_hbm, tbl_hbm, t_hbm,
             probs_hbm, tbl_v, t_v, cs_v, ns_v, tns_v, ages_v, out_v, sem):
        wid = lax.axis_index("s") * info.num_cores + lax.axis_index("c")
        pltpu.sync_copy(tbl_hbm, tbl_v)
        pltpu.sync_copy(t_hbm, t_v)
        t = t_v[...]
        my = (n_chunks - wid + nw - 1) // nw

        if tail_len:
            @pl.when(wid == nw - 1)
            def _():
                sl = pl.ds(0, tail_len)
                pltpu.sync_copy(cs_hbm.at[pl.ds(tail_off, tail_len)],
                                cs_v[0].at[sl])
                pltpu.sync_copy(ns0_hbm.at[pl.ds(tail_off, tail_len)],
                                ns_v[0].at[sl])
                pltpu.sync_copy(tns0_hbm.at[pl.ds(tail_off, tail_len)],
                                tns_v[0].at[sl])
                pltpu.sync_copy(ages_hbm.at[pl.ds(tail_off, tail_len)],
                                ages_v[0].at[sl])

                def tail_step(vi, _):
                    s2 = pl.ds(vi * 16, 16)
                    c_a = jnp.where(t >= tns_v[0][s2], ns_v[0][s2],
                                    cs_v[0][s2])
                    idx = c_a * 100 + ages_v[0][s2]
                    out_v[0][s2] = plsc.load_gather(tbl_v, [idx])
                    return 0

                lax.fori_loop(0, tail_len // 16, tail_step, 0, unroll=4)
                pltpu.sync_copy(out_v[0].at[sl],
                                probs_hbm.at[pl.ds(tail_off, tail_len)])

        def fire(k, b):
            off = (chunk_lo + wid + k * nw) * chunk
            pltpu.async_copy(cs_hbm.at[pl.ds(off, chunk)], cs_v[b], sem[b])
            pltpu.async_copy(ns0_hbm.at[pl.ds(off, chunk)], ns_v[b], sem[b])
            pltpu.async_copy(tns0_hbm.at[pl.ds(off, chunk)], tns_v[b], sem[b])
            pltpu.async_copy(ages_hbm.at[pl.ds(off, chunk)], ages_v[b], sem[b])

        def drain(b):
            pltpu.make_async_copy(cs_hbm.at[pl.ds(0, chunk)], cs_v[b],
                                  sem[b]).wait()
            pltpu.make_async_copy(ns0_hbm.at[pl.ds(0, chunk)], ns_v[b],
                                  sem[b]).wait()
            pltpu.make_async_copy(tns0_hbm.at[pl.ds(0, chunk)], tns_v[b],
                                  sem[b]).wait()
            pltpu.make_async_copy(ages_hbm.at[pl.ds(0, chunk)], ages_v[b],
                                  sem[b]).wait()

        def compute(k, b):
            def vec_step(vi, _):
                sl = pl.ds(vi * 16, 16)
                c_a = jnp.where(t >= tns_v[b][sl], ns_v[b][sl], cs_v[b][sl])
                idx = c_a * 100 + ages_v[b][sl]
                out_v[b][sl] = plsc.load_gather(tbl_v, [idx])
                return 0

            lax.fori_loop(0, chunk // 16, vec_step, 0, unroll=8)
            off = (chunk_lo + wid + k * nw) * chunk
            pltpu.sync_copy(out_v[b], probs_hbm.at[pl.ds(off, chunk)])

        @pl.when(0 < my)
        def _():
            fire(0, 0)

        def pair(j, _):
            k0 = 2 * j

            @pl.when(k0 + 1 < my)
            def _():
                fire(k0 + 1, 1)

            @pl.when(k0 < my)
            def _():
                drain(0)
                compute(k0, 0)

            @pl.when(k0 + 2 < my)
            def _():
                fire(k0 + 2, 0)

            @pl.when(k0 + 1 < my)
            def _():
                drain(1)
                compute(k0 + 1, 1)

            return 0

        lax.fori_loop(0, (n_chunks + nw - 1) // nw // 2 + 1, pair, 0)

    return sc_k(cs_p, ns0_p, tns0_p, ages_p, table_flat, t16)


def kernel(ages, current_stage, next_stage, time_to_next_stage, new_infected,
           stage_transition_probabilities, dist_mu, dist_sigma, rec_mu,
           rec_sigma, time):
    n = ages.shape[0]
    ncols = 128
    block_rows = 1024
    block_elems = block_rows * ncols
    npad = -(-n // block_elems) * block_elems
    pad = npad - n

    t = jnp.asarray(time, jnp.float32)
    t16 = jnp.broadcast_to(t, (16,))
    t1 = t.reshape(1)
    cs_i = current_stage.astype(jnp.int32)
    ns0_i = jnp.where(new_infected, 2, next_stage.astype(jnp.int32))
    tns0 = jnp.where(new_infected, t, time_to_next_stage)
    table_flat = stage_transition_probabilities.reshape(-1)

    ages_i = ages.astype(jnp.int32)
    shape2 = (npad // ncols, ncols)

    def to2d(x):
        return jnp.pad(x, (0, pad)).reshape(shape2)

    chunk = 8192
    n_blocks = npad // block_elems
    blocks0 = n_blocks // 2
    half = blocks0 * block_elems
    chunks0 = half // chunk
    chunks1 = (n - half) // chunk
    tail_off = half + chunks1 * chunk
    tail_len = n - tail_off

    probs0 = _sc_gather(cs_i, ns0_i, tns0, ages_i, table_flat, t16, npad,
                        0, chunks0, 0, 0)
    probs1 = _sc_gather(cs_i, ns0_i, tns0, ages_i, table_flat, t16, npad,
                        chunks0, chunks1, tail_off, tail_len)

    u0 = _tc_a(shape2, block_rows=block_rows)

    params = jnp.zeros((5, 8), jnp.float32)
    params = params.at[0].set(dist_mu).at[1].set(dist_sigma)
    params = params.at[2].set(rec_mu).at[3].set(rec_sigma)

    cs2, ns02, tns02 = to2d(cs_i), to2d(ns0_i), to2d(tns0)
    outs0 = _tc_b(cs2, ns02, tns02, u0, probs0.reshape(shape2), t1, params,
                  block_rows=block_rows, blk_off=0, n_blocks=blocks0)
    outs1 = _tc_b(cs2, ns02, tns02, u0, probs1.reshape(shape2), t1, params,
                  block_rows=block_rows, blk_off=blocks0,
                  n_blocks=n_blocks - blocks0)

    def assemble(a, b):
        return jnp.concatenate([a.reshape(-1)[:half],
                                b.reshape(-1)[half:n]])

    return tuple(assemble(a, b) for a, b in zip(outs0, outs1))


# speedup vs baseline: 1.0299x; 1.0299x over previous
import functools

import numpy as np
import jax
import jax.numpy as jnp
from jax import lax
from jax.experimental import pallas as pl
from jax.experimental.pallas import tpu as pltpu
from jax.experimental.pallas import tpu_sc as plsc

N_STAGES = 8


_ROT_A = (13, 15, 26, 6)
_ROT_B = (17, 29, 16, 24)


def _np_threefry(k0, k1, x0, x1):
    M = np.uint64(0xFFFFFFFF)

    def rotl(v, r):
        return ((v << np.uint64(r)) | (v >> np.uint64(32 - r))) & M

    k0 = np.uint64(k0)
    k1 = np.uint64(k1)
    ks2 = (k0 ^ k1 ^ np.uint64(0x1BD11BDA)) & M
    ks = (k0, k1, ks2)
    x0 = (np.uint64(x0) + k0) & M
    x1 = (np.uint64(x1) + k1) & M
    rots = (_ROT_A, _ROT_B)
    for g in range(5):
        for r in rots[g % 2]:
            x0 = (x0 + x1) & M
            x1 = rotl(x1, r)
            x1 ^= x0
        x0 = (x0 + ks[(g + 1) % 3]) & M
        x1 = (x1 + ks[(g + 2) % 3] + np.uint64(g + 1)) & M
    return int(x0), int(x1)


def _fold_key(i):
    return _np_threefry(0, 1234, 0, i)


_KEY_U0 = _fold_key(0)
_KEYS_SYMP = {i: _fold_key(i) for i in range(2, N_STAGES - 1)}
_KEYS_REC = {i: _fold_key(100 + i) for i in range(2, N_STAGES - 1)}

_F32_ONE_BITS = np.uint32(0x3F800000)
_U_LO = np.float32(np.nextafter(np.float32(-1.0), np.float32(0.0)))
_SQRT2 = np.float32(np.sqrt(2.0))


def _rotl(x, r):
    return (x << np.uint32(r)) | (x >> np.uint32(32 - r))


def _cipher_xor(k0, k1, ctr):
    if isinstance(k0, int):
        k0 = jnp.uint32(k0)
    if isinstance(k1, int):
        k1 = jnp.uint32(k1)
    ks2 = k0 ^ k1 ^ jnp.uint32(0x1BD11BDA)
    ks = (k0, k1, ks2)
    x0 = jnp.broadcast_to(k0, ctr.shape).astype(jnp.uint32)
    x1 = ctr + k1
    rots = (_ROT_A, _ROT_B)
    for g in range(5):
        for r in rots[g % 2]:
            x0 = x0 + x1
            x1 = _rotl(x1, r)
            x1 = x1 ^ x0
        x0 = x0 + ks[(g + 1) % 3]
        x1 = x1 + ks[(g + 2) % 3] + jnp.uint32(g + 1)
    return x0 ^ x1


def _bits_to_f01(bits):
    fb = (bits >> jnp.uint32(9)) | _F32_ONE_BITS
    return lax.bitcast_convert_type(fb, jnp.float32) - jnp.float32(1.0)


def _erfinv_f32(x):
    w = -jnp.log((jnp.float32(1.0) - x) * (jnp.float32(1.0) + x))
    w1 = w - jnp.float32(2.5)
    p = jnp.float32(2.81022636e-08)
    for c in (3.43273939e-07, -3.5233877e-06, -4.39150654e-06, 0.00021858087,
              -0.00125372503, -0.00417768164, 0.246640727, 1.50140941):
        p = jnp.float32(c) + p * w1
    w2 = jnp.sqrt(w) - jnp.float32(3.0)
    q = jnp.float32(-0.000200214257)
    for c in (0.000100950558, 0.00134934322, -0.00367342844, 0.00573950773,
              -0.0076224613, 0.00943887047, 1.00167406, 2.83297682):
        q = jnp.float32(c) + q * w2
    return jnp.where(w < jnp.float32(5.0), p, q) * x


def _chain(ist, vals, init_dtype):
    acc = jnp.full(ist.shape, vals[2], init_dtype)
    for i in range(3, N_STAGES - 1):
        acc = jnp.where(ist == i, jnp.asarray(vals[i], init_dtype), acc)
    return acc


def _ctr_for_block(shape, block_elems, blk_off=0):
    blk = pl.program_id(0).astype(jnp.uint32) + jnp.uint32(blk_off)
    row = lax.broadcasted_iota(jnp.uint32, shape, 0)
    col = lax.broadcasted_iota(jnp.uint32, shape, 1)
    return blk * jnp.uint32(block_elems) + row * jnp.uint32(shape[1]) + col


def _tca_body(block_elems, u0_out):
    ctr = _ctr_for_block(u0_out.shape, block_elems)
    u0_out[...] = jnp.maximum(
        jnp.float32(0.0),
        _bits_to_f01(_cipher_xor(_KEY_U0[0], _KEY_U0[1], ctr)))


def _tcb_body(block_elems, blk_off, t_smem, par_ref, cs_ref, ns0_ref,
              tns0_ref, u0_ref, probs_ref, *rest):
    cs1_out, ns_out, tns_out = rest[-3:]
    shape = cs_ref.shape
    t = t_smem[0]
    csf = cs_ref[...].astype(jnp.float32)
    ns0 = ns0_ref[...].astype(jnp.float32)
    tns0 = tns0_ref[...]

    mt = (t >= tns0) & (csf < jnp.float32(N_STAGES - 1))
    cs1 = jnp.where(mt, ns0, csf)
    ist = cs1.astype(jnp.int32)

    symp = u0_ref[...] < probs_ref[...]
    upd = mt & (cs1 >= jnp.float32(2.0)) & (cs1 <= jnp.float32(N_STAGES - 2))

    k0s = _chain(ist, {i: np.uint32(k[0]) for i, k in _KEYS_SYMP.items()},
                 jnp.uint32)
    k1s = _chain(ist, {i: np.uint32(k[1]) for i, k in _KEYS_SYMP.items()},
                 jnp.uint32)
    k0r = _chain(ist, {i: np.uint32(k[0]) for i, k in _KEYS_REC.items()},
                 jnp.uint32)
    k1r = _chain(ist, {i: np.uint32(k[1]) for i, k in _KEYS_REC.items()},
                 jnp.uint32)
    k0 = jnp.where(symp, k0s, k0r)
    k1 = jnp.where(symp, k1s, k1r)

    ctr = _ctr_for_block(shape, block_elems, blk_off)
    f = _bits_to_f01(_cipher_xor(k0, k1, ctr))
    u = f * jnp.float32(2.0) + _U_LO
    u = jnp.maximum(_U_LO, u)
    eps = _SQRT2 * _erfinv_f32(u)

    def chain_par(row_idx):
        acc = jnp.full(shape, par_ref[row_idx, 2])
        for i in range(3, N_STAGES - 1):
            acc = jnp.where(ist == i, par_ref[row_idx, i], acc)
        return acc

    mu = jnp.where(symp, chain_par(0), chain_par(2))
    sig = jnp.where(symp, chain_par(1), chain_par(3))
    samp = jnp.exp(mu + sig * eps)

    cs1_out[...] = cs1
    ns_out[...] = jnp.where(upd & symp, ns0 + jnp.float32(1.0),
                            jnp.where(upd, jnp.float32(0.0), ns0))
    tns_out[...] = jnp.where(upd, tns0 + samp, tns0)


def _tc_a(shape2, *, block_rows, interpret=False):
    nrows, ncols = shape2
    grid = nrows // block_rows
    bspec = pl.BlockSpec((block_rows, ncols), lambda i: (i, 0))
    f32 = jax.ShapeDtypeStruct((nrows, ncols), jnp.float32)
    return pl.pallas_call(
        functools.partial(_tca_body, block_rows * ncols),
        grid=(grid,),
        in_specs=[],
        out_specs=[bspec],
        out_shape=[f32],
        compiler_params=pltpu.CompilerParams(
            dimension_semantics=("arbitrary",)),
        interpret=interpret,
    )()[0]


def _tc_b(cs2, ns02, tns02, u0, probs2, t1, params, *, block_rows,
          blk_off=0, n_blocks=None, donate=None, interpret=False):
    nrows, ncols = cs2.shape
    grid = n_blocks if n_blocks is not None else nrows // block_rows
    bspec = pl.BlockSpec((block_rows, ncols), lambda i: (i + blk_off, 0))
    f32 = jax.ShapeDtypeStruct((nrows, ncols), jnp.float32)
    smem = pl.BlockSpec(memory_space=pltpu.SMEM)
    extra = tuple(donate) if donate else ()
    any_spec = pl.BlockSpec(memory_space=pl.ANY)
    return pl.pallas_call(
        functools.partial(_tcb_body, block_rows * ncols, blk_off),
        grid=(grid,),
        in_specs=[smem, smem, bspec, bspec, bspec, bspec, bspec] +
                 [any_spec] * len(extra),
        out_specs=[bspec, bspec, bspec],
        out_shape=[f32, f32, f32],
        input_output_aliases={7 + i: i for i in range(len(extra))},
        compiler_params=pltpu.CompilerParams(
            dimension_semantics=("arbitrary",)),
        interpret=interpret,
    )(t1, params, cs2, ns02, tns02, u0, probs2, *extra)


def _sc_gather(cs_p, ns0_p, tns0_p, ages_p, table_flat, t16, npad,
               chunk_lo, n_chunks, tail_off, tail_len):
    info = plsc.get_sparse_core_info()
    nw = info.num_cores * info.num_subcores
    chunk = 8192
    tbl_n = table_flat.shape[0]
    mesh = plsc.VectorSubcoreMesh(core_axis_name="c", subcore_axis_name="s")

    @functools.partial(
        pl.kernel, mesh=mesh,
        compiler_params=pltpu.CompilerParams(needs_layout_passes=False),
        out_type=jax.ShapeDtypeStruct((npad,), jnp.float32),
        scratch_types=[
            pltpu.VMEM((tbl_n,), jnp.float32),
            pltpu.VMEM((16,), jnp.float32),
            [pltpu.VMEM((chunk,), jnp.int32) for _ in range(2)],
            [pltpu.VMEM((chunk,), jnp.int32) for _ in range(2)],
            [pltpu.VMEM((chunk,), jnp.float32) for _ in range(2)],
            [pltpu.VMEM((chunk,), jnp.int32) for _ in range(2)],
            [pltpu.VMEM((chunk,), jnp.float32) for _ in range(2)],
            [pltpu.SemaphoreType.DMA for _ in range(2)],
        ],
    )
    def sc_k(cs_hbm, ns0_hbm, tns0_hbm, ages_hbm, tbl_hbm, t_hbm,
             probs_hbm, tbl_v, t_v, cs_v, ns_v, tns_v, ages_v, out_v, sem):
        wid = lax.axis_index("s") * info.num_cores + lax.axis_index("c")
        pltpu.sync_copy(tbl_hbm, tbl_v)
        pltpu.sync_copy(t_hbm, t_v)
        t = t_v[...]
        my = (n_chunks - wid + nw - 1) // nw

        if tail_len:
            @pl.when(wid == nw - 1)
            def _():
                sl = pl.ds(0, tail_len)
                pltpu.sync_copy(cs_hbm.at[pl.ds(tail_off, tail_len)],
                                cs_v[0].at[sl])
                pltpu.sync_copy(ns0_hbm.at[pl.ds(tail_off, tail_len)],
                                ns_v[0].at[sl])
                pltpu.sync_copy(tns0_hbm.at[pl.ds(tail_off, tail_len)],
                                tns_v[0].at[sl])
                pltpu.sync_copy(ages_hbm.at[pl.ds(tail_off, tail_len)],
                                ages_v[0].at[sl])

                def tail_step(vi, _):
                    s2 = pl.ds(vi * 16, 16)
                    c_a = jnp.where(t >= tns_v[0][s2], ns_v[0][s2],
                                    cs_v[0][s2])
                    idx = c_a * 100 + ages_v[0][s2]
                    out_v[0][s2] = plsc.load_gather(tbl_v, [idx])
                    return 0

                lax.fori_loop(0, tail_len // 16, tail_step, 0, unroll=4)
                pltpu.sync_copy(out_v[0].at[sl],
                                probs_hbm.at[pl.ds(tail_off, tail_len)])

        def fire(k, b):
            off = (chunk_lo + wid + k * nw) * chunk
            pltpu.async_copy(cs_hbm.at[pl.ds(off, chunk)], cs_v[b], sem[b])
            pltpu.async_copy(ns0_hbm.at[pl.ds(off, chunk)], ns_v[b], sem[b])
            pltpu.async_copy(tns0_hbm.at[pl.ds(off, chunk)], tns_v[b], sem[b])
            pltpu.async_copy(ages_hbm.at[pl.ds(off, chunk)], ages_v[b], sem[b])

        def drain(b):
            pltpu.make_async_copy(cs_hbm.at[pl.ds(0, chunk)], cs_v[b],
                                  sem[b]).wait()
            pltpu.make_async_copy(ns0_hbm.at[pl.ds(0, chunk)], ns_v[b],
                                  sem[b]).wait()
            pltpu.make_async_copy(tns0_hbm.at[pl.ds(0, chunk)], tns_v[b],
                                  sem[b]).wait()
            pltpu.make_async_copy(ages_hbm.at[pl.ds(0, chunk)], ages_v[b],
                                  sem[b]).wait()

        def compute(k, b):
            def vec_step(vi, _):
                sl = pl.ds(vi * 16, 16)
                c_a = jnp.where(t >= tns_v[b][sl], ns_v[b][sl], cs_v[b][sl])
                idx = c_a * 100 + ages_v[b][sl]
                out_v[b][sl] = plsc.load_gather(tbl_v, [idx])
                return 0

            lax.fori_loop(0, chunk // 16, vec_step, 0, unroll=8)
            off = (chunk_lo + wid + k * nw) * chunk
            pltpu.sync_copy(out_v[b], probs_hbm.at[pl.ds(off, chunk)])

        @pl.when(0 < my)
        def _():
            fire(0, 0)

        def pair(j, _):
            k0 = 2 * j

            @pl.when(k0 + 1 < my)
            def _():
                fire(k0 + 1, 1)

            @pl.when(k0 < my)
            def _():
                drain(0)
                compute(k0, 0)

            @pl.when(k0 + 2 < my)
            def _():
                fire(k0 + 2, 0)

            @pl.when(k0 + 1 < my)
            def _():
                drain(1)
                compute(k0 + 1, 1)

            return 0

        lax.fori_loop(0, (n_chunks + nw - 1) // nw // 2 + 1, pair, 0)

    return sc_k(cs_p, ns0_p, tns0_p, ages_p, table_flat, t16)


def kernel(ages, current_stage, next_stage, time_to_next_stage, new_infected,
           stage_transition_probabilities, dist_mu, dist_sigma, rec_mu,
           rec_sigma, time):
    n = ages.shape[0]
    ncols = 128
    block_rows = 1024
    block_elems = block_rows * ncols
    npad = -(-n // block_elems) * block_elems
    pad = npad - n

    t = jnp.asarray(time, jnp.float32)
    t16 = jnp.broadcast_to(t, (16,))
    t1 = t.reshape(1)
    cs_i = current_stage.astype(jnp.int32)
    ns0_i = jnp.where(new_infected, 2, next_stage.astype(jnp.int32))
    tns0 = jnp.where(new_infected, t, time_to_next_stage)
    table_flat = stage_transition_probabilities.reshape(-1)

    ages_i = ages.astype(jnp.int32)
    shape2 = (npad // ncols, ncols)

    def to2d(x):
        return jnp.pad(x, (0, pad)).reshape(shape2)

    chunk = 8192
    n_blocks = npad // block_elems
    blocks0 = n_blocks // 2
    half = blocks0 * block_elems
    chunks0 = half // chunk
    chunks1 = (n - half) // chunk
    tail_off = half + chunks1 * chunk
    tail_len = n - tail_off

    probs0 = _sc_gather(cs_i, ns0_i, tns0, ages_i, table_flat, t16, npad,
                        0, chunks0, 0, 0)
    probs1 = _sc_gather(cs_i, ns0_i, tns0, ages_i, table_flat, t16, npad,
                        chunks0, chunks1, tail_off, tail_len)

    u0 = _tc_a(shape2, block_rows=block_rows)

    params = jnp.zeros((5, 8), jnp.float32)
    params = params.at[0].set(dist_mu).at[1].set(dist_sigma)
    params = params.at[2].set(rec_mu).at[3].set(rec_sigma)

    cs2, ns02, tns02 = to2d(cs_i), to2d(ns0_i), to2d(tns0)
    outs0 = _tc_b(cs2, ns02, tns02, u0, probs0.reshape(shape2), t1, params,
                  block_rows=block_rows, blk_off=0, n_blocks=blocks0)
    outs1 = _tc_b(cs2, ns02, tns02, u0, probs1.reshape(shape2), t1, params,
                  block_rows=block_rows, blk_off=blocks0,
                  n_blocks=n_blocks - blocks0, donate=outs0)

    return tuple(o.reshape(-1)[:n] for o in outs1)
